# baseline (speedup 1.0000x reference)
import jax
import jax.numpy as jnp
from jax.experimental import pallas as pl
from jax.experimental.pallas import tpu as pltpu

N_BLK = 2000
MP = 128
POS_THR = 0.15
NEG_THR = 0.7
BIGI = 2 ** 30


def _body(a_ref, gt_ref, gtc_ref, lab_ref, box_ref, vscr, iscr, rmscr):
    i = pl.program_id(0)
    j = pl.program_id(1)
    B = a_ref.shape[0]
    rowid = jax.lax.broadcasted_iota(jnp.int32, (B, MP), 0) + j * B

    @pl.when(i == 0)
    def _pass0():
        a = a_ref[...]
        ax1 = a[:, 0:1]
        ay1 = a[:, 1:2]
        ax2 = a[:, 2:3]
        ay2 = a[:, 3:4]
        ltx = jnp.maximum(ax1, gt_ref[0:1, :])
        lty = jnp.maximum(ay1, gt_ref[1:2, :])
        rbx = jnp.minimum(ax2, gt_ref[2:3, :])
        rby = jnp.minimum(ay2, gt_ref[3:4, :])
        whx = jnp.maximum(rbx - ltx, 0.0)
        why = jnp.maximum(rby - lty, 0.0)
        inter = whx * why
        areaa = (ax2 - ax1) * (ay2 - ay1)
        union = (areaa + gt_ref[4:5, :]) - inter
        iou = inter / jnp.maximum(union, 1e-6)

        rmscr[pl.ds(j * B, B), :] = jnp.max(iou, axis=1, keepdims=True)

        @pl.when(j == 0)
        def _init():
            vscr[...] = jnp.full((8, MP), -1.0, jnp.float32)
            iscr[...] = jnp.full((8, MP), BIGI, jnp.int32)

        cur = iou
        bv = []
        bi = []
        for t in range(4):
            cmax = jnp.max(cur, axis=0, keepdims=True)
            cand = jnp.where(cur == cmax, rowid, BIGI)
            cidx = jnp.min(cand, axis=0, keepdims=True)
            bv.append(cmax)
            bi.append(cidx)
            if t < 3:
                cur = jnp.where(rowid == cidx, -1.0, cur)
        comb_v = jnp.concatenate([vscr[0:4, :]] + bv, axis=0)
        comb_i = jnp.concatenate([iscr[0:4, :]] + bi, axis=0)
        nv = []
        ni = []
        for t in range(4):
            cmax = jnp.max(comb_v, axis=0, keepdims=True)
            cand = jnp.where(comb_v == cmax, comb_i, BIGI)
            cidx = jnp.min(cand, axis=0, keepdims=True)
            nv.append(cmax)
            ni.append(cidx)
            if t < 3:
                comb_v = jnp.where(comb_i == cidx, -2.0, comb_v)
        vscr[...] = jnp.concatenate(nv + nv, axis=0)
        iscr[...] = jnp.concatenate(ni + ni, axis=0)

    @pl.when(i == 1)
    def _pass1():
        mplus1 = jax.lax.broadcasted_iota(jnp.int32, (1, MP), 1) + 1
        idxf = jnp.where(vscr[0:4, :] >= POS_THR, iscr[0:4, :], -5)
        match = rowid == idxf[0:1, :]
        for t in range(1, 4):
            match = match | (rowid == idxf[t:t + 1, :])
        am = jnp.max(jnp.where(match, mplus1, -1), axis=1, keepdims=True)
        rm = rmscr[pl.ds(j * B, B), :]
        assigned = jnp.where(am == -1, jnp.where(rm < NEG_THR, 0, -1), am)
        pos = assigned > 0
        neg = assigned == 0
        onehot = (assigned == mplus1).astype(jnp.float32)
        tab = jax.lax.dot_general(
            onehot, gtc_ref[...], (((1,), (0,)), ((), ())),
            precision=jax.lax.Precision.HIGHEST,
            preferred_element_type=jnp.float32)
        lab_ref[...] = jnp.where(
            pos, tab[:, 4:5].astype(jnp.int32), jnp.where(neg, 0, -1))
        box_ref[...] = jnp.where(pos, tab[:, 0:4], -1.0)


def kernel(grid_bboxes, gt_bboxes, gt_labels):
    N = grid_bboxes.shape[0]
    M = gt_bboxes.shape[0]
    pad = jnp.full((MP - M, 4), -1e9, jnp.float32)
    gtp = jnp.concatenate([gt_bboxes, pad], axis=0)
    area_b = (gtp[:, 2] - gtp[:, 0]) * (gtp[:, 3] - gtp[:, 1])
    labp = jnp.concatenate(
        [gt_labels.astype(jnp.float32), jnp.zeros((MP - M,), jnp.float32)])
    zeros = jnp.zeros((MP,), jnp.float32)
    gtT = jnp.stack(
        [gtp[:, 0], gtp[:, 1], gtp[:, 2], gtp[:, 3], area_b, labp, zeros, zeros],
        axis=0)
    gtC = jnp.stack(
        [gtp[:, 0], gtp[:, 1], gtp[:, 2], gtp[:, 3], labp, zeros, zeros, zeros],
        axis=1)
    nb = N // N_BLK

    lab2, boxes = pl.pallas_call(
        _body,
        grid=(2, nb),
        in_specs=[
            pl.BlockSpec((N_BLK, 4), lambda i, j: (j, 0)),
            pl.BlockSpec((8, MP), lambda i, j: (0, 0)),
            pl.BlockSpec((MP, 8), lambda i, j: (0, 0)),
        ],
        out_specs=[
            pl.BlockSpec((N_BLK, 1), lambda i, j: (j, 0)),
            pl.BlockSpec((N_BLK, 4), lambda i, j: (j, 0)),
        ],
        out_shape=[
            jax.ShapeDtypeStruct((N, 1), jnp.int32),
            jax.ShapeDtypeStruct((N, 4), jnp.float32),
        ],
        scratch_shapes=[
            pltpu.VMEM((8, MP), jnp.float32),
            pltpu.VMEM((8, MP), jnp.int32),
            pltpu.VMEM((N, 1), jnp.float32),
        ],
    )(grid_bboxes, gtT, gtC)

    return lab2.reshape(N), boxes
